# baseline (speedup 1.0000x reference)
import functools

import jax
import jax.numpy as jnp
import numpy as _np
from jax import lax
from jax.experimental import pallas as pl
from jax.experimental.pallas import tpu as pltpu
from jax.experimental.pallas import tpu_sc as plsc

_N = 10000
_E = 320000
_D = 128
_K = 32

_NC = 2
_NS = 16
_NW = _NC * _NS
_CW = 125
_EPW = _E // _NW
_NCH = _EPW // _CW
_EROWS = _E // _CW
_NP = 10240
_SLC = _NP // _NS
_RB = 2000


def _deg_body(src_hbm, dst_hbm, out_hbm, idx_v, val_v, z_v, od_sh, id_sh,
              sa):
    c = lax.axis_index("c")
    s = lax.axis_index("s")
    w = c * _NS + s

    zv = jnp.zeros((16,), jnp.float32)

    def zfill(i, carry):
        z_v[pl.ds(i * 16, 16)] = zv
        return carry

    lax.fori_loop(0, _SLC // 16, zfill, 0)
    ov = jnp.full((16,), 1.0, jnp.float32)
    for j in range(8):
        val_v[pl.ds(j * 16, 16)] = ov

    pltpu.sync_copy(z_v, od_sh.at[pl.ds(s * _SLC, _SLC)])
    pltpu.sync_copy(z_v, id_sh.at[pl.ds(s * _SLC, _SLC)])
    plsc.subcore_barrier()

    val = val_v.at[pl.ds(0, _CW)]

    def scat_all(edge_hbm, tgt_sh):
        pltpu.sync_copy(edge_hbm.at[pl.ds(w * _NCH, _NCH)], idx_v)

        def body(k, carry):
            pltpu.make_async_copy(val, tgt_sh.at[idx_v.at[k]],
                                  sa).start(add=True)

            @pl.when(k >= 4)
            def _():
                pltpu.make_async_copy(val, tgt_sh.at[idx_v.at[0]], sa).wait()

            return carry

        lax.fori_loop(0, _NCH, body, 0)
        for _i in range(4):
            pltpu.make_async_copy(val, tgt_sh.at[idx_v.at[0]], sa).wait()

    scat_all(src_hbm, od_sh)
    scat_all(dst_hbm, id_sh)

    plsc.subcore_barrier()
    pltpu.sync_copy(od_sh.at[pl.ds(s * _SLC, _SLC)],
                    out_hbm.at[pl.ds(c * _NP + s * _SLC, _SLC)])
    pltpu.sync_copy(id_sh.at[pl.ds(s * _SLC, _SLC)],
                    out_hbm.at[pl.ds((2 + c) * _NP + s * _SLC, _SLC)])


@functools.cache
def _deg_kernel():
    return pl.kernel(
        _deg_body,
        out_type=jax.ShapeDtypeStruct((4 * _NP,), jnp.float32),
        mesh=plsc.VectorSubcoreMesh(
            core_axis_name="c", subcore_axis_name="s",
            num_cores=_NC, num_subcores=_NS),
        scratch_types=[
            pltpu.VMEM((_NCH, _CW), jnp.int32),
            pltpu.VMEM((128,), jnp.float32),
            pltpu.VMEM((_SLC,), jnp.float32),
            pltpu.VMEM_SHARED((_NP,), jnp.float32),
            pltpu.VMEM_SHARED((_NP,), jnp.float32),
            pltpu.SemaphoreType.DMA,
        ],
    )


_HD = _D // 2
_NB = 4
_CPT = _E // _CW // _NS
_PH = 4
_CPP = _CPT // _PH


def _agg_body(src_hbm, dst_hbm, h_hbm, out_hbm, idxs_v, idxd_v, rows_v,
              acc_sh, sg, ss):
    c = lax.axis_index("c")
    s = lax.axis_index("s")

    zv = jnp.zeros((16,), jnp.float32)

    def zfill(i, carry):
        rows_v[0, i // 4, pl.ds((i % 4) * 16, 16)] = zv
        return carry

    lax.fori_loop(0, 80 * 4, zfill, 0)
    for i in range(_SLC // 80):
        pltpu.sync_copy(rows_v.at[0, pl.ds(0, 80)],
                        acc_sh.at[pl.ds(s * _SLC + i * 80, 80)])
    plsc.subcore_barrier()

    def g_desc(k, b):
        return pltpu.make_async_copy(h_hbm.at[c].at[idxs_v.at[k]], rows_v.at[b],
                                     sg.at[b])

    def s_desc(k, b):
        return pltpu.make_async_copy(rows_v.at[b], acc_sh.at[idxd_v.at[k]],
                                     ss.at[b])

    for p in range(_PH):
        base = s * _CPT + p * _CPP
        pltpu.sync_copy(src_hbm.at[pl.ds(base, _CPP)], idxs_v)
        pltpu.sync_copy(dst_hbm.at[pl.ds(base, _CPP)], idxd_v)

        for k in range(_NB):
            g_desc(k, k).start()
        g_desc(0, 0).wait()
        s_desc(0, 0).start(add=True)

        def body(k, carry):
            b = lax.rem(k, _NB)
            s_desc(k - 1, lax.rem(k - 1 + _NB, _NB)).wait()
            g_desc(k + _NB - 1, lax.rem(k + _NB - 1, _NB)).start()
            g_desc(k, b).wait()
            s_desc(k, b).start(add=True)
            return carry

        lax.fori_loop(1, _CPP - _NB + 1, body, 0)

        for k in range(_CPP - _NB + 1, _CPP):
            b = k % _NB
            s_desc(k - 1, (k - 1) % _NB).wait()
            g_desc(k, b).wait()
            s_desc(k, b).start(add=True)
        s_desc(_CPP - 1, (_CPP - 1) % _NB).wait()

    plsc.subcore_barrier()
    pltpu.sync_copy(acc_sh.at[pl.ds(s * _SLC, _SLC)],
                    out_hbm.at[c, pl.ds(s * _SLC, _SLC)])


@functools.cache
def _agg_kernel():
    return pl.kernel(
        _agg_body,
        out_type=jax.ShapeDtypeStruct((_NC, _NP, _HD), jnp.float32),
        mesh=plsc.VectorSubcoreMesh(
            core_axis_name="c", subcore_axis_name="s",
            num_cores=_NC, num_subcores=_NS),
        scratch_types=[
            pltpu.VMEM((_CPP, _CW), jnp.int32),
            pltpu.VMEM((_CPP, _CW), jnp.int32),
            pltpu.VMEM((_NB, _CW, _HD), jnp.float32),
            pltpu.VMEM_SHARED((_NP, _HD), jnp.float32),
            pltpu.SemaphoreType.DMA((_NB,)),
            pltpu.SemaphoreType.DMA((_NB,)),
        ],
        compiler_params=pltpu.CompilerParams(use_tc_tiling_on_sc=False),
    )


def _bitonic_stages():
    stages = []
    k = 2
    while k <= _D:
        j = k // 2
        while j >= 1:
            stages.append((k, j))
            j //= 2
        k *= 2
    return stages


_BSTAGES = _bitonic_stages()


def _proj_body(f_ref, w_ref, d_ref, o_ref):
    h = jnp.dot(f_ref[...], w_ref[...], preferred_element_type=jnp.float32)

    lane = lax.broadcasted_iota(jnp.int32, (1, _D), 1)
    s = h
    for k, j in _BSTAGES:
        hi = (lane & j) != 0
        partner = jnp.where(hi, pltpu.roll(s, j, 1),
                            pltpu.roll(s, _D - j, 1))
        mask_min = ((lane & k) == 0) == jnp.logical_not(hi)
        s = jnp.where(mask_min, jnp.minimum(s, partner),
                      jnp.maximum(s, partner))
    thr = lax.slice(s, (0, _D - _K), (_RB, _D - _K + 1))
    keep = h >= thr

    dg = d_ref[...]
    od = jnp.maximum(dg[:, 0:1] + dg[:, 1:2], 1.0)
    ind = jnp.maximum(dg[:, 2:3] + dg[:, 3:4], 1.0)
    scale = lax.rsqrt(od * ind)
    hs = jnp.where(keep, h, 0.0) * scale
    o_ref[0] = lax.slice(hs, (0, 0), (_RB, _HD))
    o_ref[1] = lax.slice(hs, (0, _HD), (_RB, _D))


_proj_kernel = pl.pallas_call(
    _proj_body,
    grid=(_N // _RB,),
    in_specs=[
        pl.BlockSpec((_RB, _D), lambda i: (i, 0)),
        pl.BlockSpec((_D, _D), lambda i: (0, 0)),
        pl.BlockSpec((_RB, 4), lambda i: (i, 0)),
    ],
    out_specs=pl.BlockSpec((2, _RB, _HD), lambda i: (0, i, 0)),
    out_shape=jax.ShapeDtypeStruct((2, _N, _HD), jnp.float32),
)


def _merge_body(p_ref, b_ref, o_ref):
    both = jnp.concatenate([p_ref[0], p_ref[1]], axis=1)
    o_ref[...] = both + b_ref[0]


_merge_kernel = pl.pallas_call(
    _merge_body,
    grid=(_N // _RB,),
    in_specs=[
        pl.BlockSpec((_NC, _RB, _HD), lambda i: (0, i, 0)),
        pl.BlockSpec((1, _D), lambda i: (0, 0)),
    ],
    out_specs=pl.BlockSpec((_RB, _D), lambda i: (i, 0)),
    out_shape=jax.ShapeDtypeStruct((_N, _D), jnp.float32),
)


def kernel(edge_index, feat, W, bias):
    src2d = edge_index[0].astype(jnp.int32).reshape(_EROWS, _CW)
    dst2d = edge_index[1].astype(jnp.int32).reshape(_EROWS, _CW)
    degp = _deg_kernel()(src2d, dst2d)
    degs = degp.reshape(4, _NP)[:, :_N].T
    hs = _proj_kernel(feat, W, degs)
    parts = _agg_kernel()(src2d, dst2d, hs)
    return _merge_kernel(parts, bias.reshape(1, _D))


# speedup vs baseline: 1.0409x; 1.0409x over previous
import functools

import jax
import jax.numpy as jnp
import numpy as _np
from jax import lax
from jax.experimental import pallas as pl
from jax.experimental.pallas import tpu as pltpu
from jax.experimental.pallas import tpu_sc as plsc

_N = 10000
_E = 320000
_D = 128
_K = 32

_NC = 2
_NS = 16
_NW = _NC * _NS
_CW = 125
_EPW = _E // _NW
_NCH = _EPW // _CW
_EROWS = _E // _CW
_NP = 10240
_SLC = _NP // _NS
_RB = 2000


def _deg_body(src_hbm, dst_hbm, out_hbm, idx_v, val_v, z_v, od_sh, id_sh,
              sa):
    c = lax.axis_index("c")
    s = lax.axis_index("s")
    w = c * _NS + s

    zv = jnp.zeros((16,), jnp.float32)

    def zfill(i, carry):
        z_v[pl.ds(i * 16, 16)] = zv
        return carry

    lax.fori_loop(0, _SLC // 16, zfill, 0)
    ov = jnp.full((16,), 1.0, jnp.float32)
    for j in range(8):
        val_v[pl.ds(j * 16, 16)] = ov

    pltpu.sync_copy(z_v, od_sh.at[pl.ds(s * _SLC, _SLC)])
    pltpu.sync_copy(z_v, id_sh.at[pl.ds(s * _SLC, _SLC)])
    plsc.subcore_barrier()

    val = val_v.at[pl.ds(0, _CW)]

    def scat_all(edge_hbm, tgt_sh):
        pltpu.sync_copy(edge_hbm.at[pl.ds(w * _NCH, _NCH)], idx_v)

        def body(k, carry):
            pltpu.make_async_copy(val, tgt_sh.at[idx_v.at[k]],
                                  sa).start(add=True)

            @pl.when(k >= 4)
            def _():
                pltpu.make_async_copy(val, tgt_sh.at[idx_v.at[0]], sa).wait()

            return carry

        lax.fori_loop(0, _NCH, body, 0)
        for _i in range(4):
            pltpu.make_async_copy(val, tgt_sh.at[idx_v.at[0]], sa).wait()

    scat_all(src_hbm, od_sh)
    scat_all(dst_hbm, id_sh)

    plsc.subcore_barrier()
    pltpu.sync_copy(od_sh.at[pl.ds(s * _SLC, _SLC)],
                    out_hbm.at[pl.ds(c * _NP + s * _SLC, _SLC)])
    pltpu.sync_copy(id_sh.at[pl.ds(s * _SLC, _SLC)],
                    out_hbm.at[pl.ds((2 + c) * _NP + s * _SLC, _SLC)])


@functools.cache
def _deg_kernel():
    return pl.kernel(
        _deg_body,
        out_type=jax.ShapeDtypeStruct((4 * _NP,), jnp.float32),
        mesh=plsc.VectorSubcoreMesh(
            core_axis_name="c", subcore_axis_name="s",
            num_cores=_NC, num_subcores=_NS),
        scratch_types=[
            pltpu.VMEM((_NCH, _CW), jnp.int32),
            pltpu.VMEM((128,), jnp.float32),
            pltpu.VMEM((_SLC,), jnp.float32),
            pltpu.VMEM_SHARED((_NP,), jnp.float32),
            pltpu.VMEM_SHARED((_NP,), jnp.float32),
            pltpu.SemaphoreType.DMA,
        ],
    )


_PH = 2
_CPP = _NCH // _PH


def _agg_body(src_hbm, dst_hbm, h_hbm, out_hbm, idxs_v, idxd_v, rows_v,
              acc_sh, sg, ss):
    c = lax.axis_index("c")
    s = lax.axis_index("s")
    w = c * _NS + s

    zv = jnp.zeros((16,), jnp.float32)

    def zfill(i, carry):
        rows_v[0, i // 8, pl.ds((i % 8) * 16, 16)] = zv
        return carry

    lax.fori_loop(0, 80 * 8, zfill, 0)
    for i in range(_SLC // 80):
        pltpu.sync_copy(rows_v.at[0, pl.ds(0, 80)],
                        acc_sh.at[pl.ds(s * _SLC + i * 80, 80)])
    plsc.subcore_barrier()

    def g_desc(k, b):
        return pltpu.make_async_copy(h_hbm.at[idxs_v.at[k]], rows_v.at[b],
                                     sg.at[b])

    def s_desc(k, b):
        return pltpu.make_async_copy(rows_v.at[b], acc_sh.at[idxd_v.at[k]],
                                     ss.at[b])

    for p in range(_PH):
        base = w * _NCH + p * _CPP
        pltpu.sync_copy(src_hbm.at[pl.ds(base, _CPP)], idxs_v)
        pltpu.sync_copy(dst_hbm.at[pl.ds(base, _CPP)], idxd_v)

        g_desc(0, 0).start()
        g_desc(1, 1).start()
        g_desc(0, 0).wait()
        s_desc(0, 0).start(add=True)

        def body(k, carry):
            b = lax.rem(k, 2)
            bn = 1 - b
            s_desc(k - 1, bn).wait()
            g_desc(k + 1, bn).start()
            g_desc(k, b).wait()
            s_desc(k, b).start(add=True)
            return carry

        lax.fori_loop(1, _CPP - 1, body, 0)

        g_desc(_CPP - 1, 1).wait()
        s_desc(_CPP - 1, 1).start(add=True)
        s_desc(_CPP - 2, 0).wait()
        s_desc(_CPP - 1, 1).wait()

    plsc.subcore_barrier()
    pltpu.sync_copy(acc_sh.at[pl.ds(s * _SLC, _SLC)],
                    out_hbm.at[c, pl.ds(s * _SLC, _SLC)])


@functools.cache
def _agg_kernel():
    return pl.kernel(
        _agg_body,
        out_type=jax.ShapeDtypeStruct((_NC, _NP, _D), jnp.float32),
        mesh=plsc.VectorSubcoreMesh(
            core_axis_name="c", subcore_axis_name="s",
            num_cores=_NC, num_subcores=_NS),
        scratch_types=[
            pltpu.VMEM((_CPP, _CW), jnp.int32),
            pltpu.VMEM((_CPP, _CW), jnp.int32),
            pltpu.VMEM((2, _CW, _D), jnp.float32),
            pltpu.VMEM_SHARED((_NP, _D), jnp.float32),
            pltpu.SemaphoreType.DMA((2,)),
            pltpu.SemaphoreType.DMA((2,)),
        ],
    )


def _bitonic_stages():
    stages = []
    k = 2
    while k <= _D:
        j = k // 2
        while j >= 1:
            stages.append((k, j))
            j //= 2
        k *= 2
    return stages


_BSTAGES = _bitonic_stages()


def _proj_body(f_ref, w_ref, d_ref, o_ref):
    h = jnp.dot(f_ref[...], w_ref[...], preferred_element_type=jnp.float32)

    lane = lax.broadcasted_iota(jnp.int32, (1, _D), 1)
    s = h
    for k, j in _BSTAGES:
        hi = (lane & j) != 0
        partner = jnp.where(hi, pltpu.roll(s, j, 1),
                            pltpu.roll(s, _D - j, 1))
        mask_min = ((lane & k) == 0) == jnp.logical_not(hi)
        s = jnp.where(mask_min, jnp.minimum(s, partner),
                      jnp.maximum(s, partner))
    thr = lax.slice(s, (0, _D - _K), (_RB, _D - _K + 1))
    keep = h >= thr

    dg = d_ref[...]
    od = jnp.maximum(dg[:, 0:1] + dg[:, 1:2], 1.0)
    ind = jnp.maximum(dg[:, 2:3] + dg[:, 3:4], 1.0)
    scale = lax.rsqrt(od * ind)
    o_ref[...] = jnp.where(keep, h, 0.0) * scale


_proj_kernel = pl.pallas_call(
    _proj_body,
    grid=(_N // _RB,),
    in_specs=[
        pl.BlockSpec((_RB, _D), lambda i: (i, 0)),
        pl.BlockSpec((_D, _D), lambda i: (0, 0)),
        pl.BlockSpec((_RB, 4), lambda i: (i, 0)),
    ],
    out_specs=pl.BlockSpec((_RB, _D), lambda i: (i, 0)),
    out_shape=jax.ShapeDtypeStruct((_N, _D), jnp.float32),
)


def _merge_body(p_ref, b_ref, o_ref):
    o_ref[...] = p_ref[0] + p_ref[1] + b_ref[0]


_merge_kernel = pl.pallas_call(
    _merge_body,
    grid=(_N // _RB,),
    in_specs=[
        pl.BlockSpec((_NC, _RB, _D), lambda i: (0, i, 0)),
        pl.BlockSpec((1, _D), lambda i: (0, 0)),
    ],
    out_specs=pl.BlockSpec((_RB, _D), lambda i: (i, 0)),
    out_shape=jax.ShapeDtypeStruct((_N, _D), jnp.float32),
)


def kernel(edge_index, feat, W, bias):
    src2d = edge_index[0].astype(jnp.int32).reshape(_EROWS, _CW)
    dst2d = edge_index[1].astype(jnp.int32).reshape(_EROWS, _CW)
    degp = _deg_kernel()(src2d, dst2d)
    degs = degp.reshape(4, _NP)[:, :_N].T
    hs = _proj_kernel(feat, W, degs)
    parts = _agg_kernel()(src2d, dst2d, hs)
    return _merge_kernel(parts, bias.reshape(1, _D))
